# baseline (speedup 1.0000x reference)
import jax
import jax.numpy as jnp
from jax.experimental import pallas as pl
from jax.experimental.pallas import tpu as pltpu

_D = 1024
_E = 16
_K = 8
_F = 2 * _D


def _fused_body(x_ref, gw1_ref, gb1_ref, gw2_ref, gb2_ref,
                we1_hbm, be1_hbm, we2_hbm, be2_hbm, lw_hbm, lb_hbm,
                probs_ref, out_ref,
                w1buf, w2buf, b1buf, b2buf, lwbuf, lbbuf, sems):
    x = x_ref[...]
    h = jnp.maximum(
        jnp.dot(x, gw1_ref[...], preferred_element_type=jnp.float32)
        + gb1_ref[...], 0.0)
    s = jnp.dot(h, gw2_ref[...], preferred_element_type=jnp.float32) \
        + gb2_ref[...]
    m = jnp.max(s, axis=1, keepdims=True)
    e = jnp.exp(s - m)
    probs = e / jnp.sum(e, axis=1, keepdims=True)
    probs_ref[...] = probs

    iota_e = jax.lax.broadcasted_iota(jnp.int32, (1, _E), 1)
    iota_k = jax.lax.broadcasted_iota(jnp.int32, (1, _K), 1)
    p = probs
    vals = jnp.zeros((1, _K), jnp.float32)
    ais = []
    for i in range(_K):
        mv = jnp.max(p, axis=1, keepdims=True)
        ai = jnp.min(jnp.where(p == mv, iota_e, _E))
        vals = jnp.where(iota_k == i, mv, vals)
        ais.append(ai)
        p = jnp.where(iota_e == ai, -jnp.inf, p)
    vm = jnp.max(vals, axis=1, keepdims=True)
    ev = jnp.exp(vals - vm)
    gates = ev / jnp.sum(ev, axis=1, keepdims=True)

    def copies(slot, eidx):
        return [
            pltpu.make_async_copy(we1_hbm.at[eidx], w1buf.at[slot],
                                  sems.at[0, slot]),
            pltpu.make_async_copy(we2_hbm.at[eidx], w2buf.at[slot],
                                  sems.at[1, slot]),
            pltpu.make_async_copy(be1_hbm.at[eidx], b1buf.at[slot],
                                  sems.at[2, slot]),
            pltpu.make_async_copy(be2_hbm.at[eidx], b2buf.at[slot],
                                  sems.at[3, slot]),
            pltpu.make_async_copy(lw_hbm.at[eidx], lwbuf.at[slot],
                                  sems.at[4, slot]),
            pltpu.make_async_copy(lb_hbm.at[eidx], lbbuf.at[slot],
                                  sems.at[5, slot]),
        ]

    for c in copies(0, ais[0]):
        c.start()

    acc = jnp.zeros((1, _D), jnp.float32)
    for k in range(_K):
        slot = k % 2
        if k + 1 < _K:
            for c in copies((k + 1) % 2, ais[k + 1]):
                c.start()
        cs = copies(slot, ais[k])
        cs[0].wait()
        cs[2].wait()
        hh = jnp.dot(x, w1buf[slot], preferred_element_type=jnp.float32) \
            + b1buf[slot]
        hh = 0.5 * hh * (1.0 + jax.lax.erf(hh * 0.7071067811865476))
        cs[1].wait()
        cs[3].wait()
        cs[4].wait()
        cs[5].wait()
        oo = jnp.dot(hh, w2buf[slot], preferred_element_type=jnp.float32) \
            + b2buf[slot]
        mu = jnp.mean(oo, axis=1, keepdims=True)
        d = oo - mu
        var = jnp.mean(d * d, axis=1, keepdims=True)
        nn = d * jax.lax.rsqrt(var + 1e-5) * lwbuf[slot] + lbbuf[slot]
        acc = acc + gates[:, k:k + 1] * nn
    out_ref[...] = acc


@jax.jit
def kernel(features, gate_W1, gate_b1, gate_W2, gate_b2,
           We1, be1, We2, be2, ln_w, ln_b):
    x = features.reshape(-1)[:_D].reshape(1, _D)
    hbm = pl.BlockSpec(memory_space=pltpu.MemorySpace.HBM)
    probs, out = pl.pallas_call(
        _fused_body,
        in_specs=[
            pl.BlockSpec((1, _D), lambda: (0, 0)),
            pl.BlockSpec((_D, _D // 2), lambda: (0, 0)),
            pl.BlockSpec((1, _D // 2), lambda: (0, 0)),
            pl.BlockSpec((_D // 2, _E), lambda: (0, 0)),
            pl.BlockSpec((1, _E), lambda: (0, 0)),
            hbm, hbm, hbm, hbm, hbm, hbm,
        ],
        out_specs=(
            pl.BlockSpec((1, _E), lambda: (0, 0)),
            pl.BlockSpec((1, _D), lambda: (0, 0)),
        ),
        out_shape=(
            jax.ShapeDtypeStruct((1, _E), jnp.float32),
            jax.ShapeDtypeStruct((1, _D), jnp.float32),
        ),
        scratch_shapes=[
            pltpu.VMEM((2, _D, _F), jnp.float32),
            pltpu.VMEM((2, _F, _D), jnp.float32),
            pltpu.VMEM((2, 1, _F), jnp.float32),
            pltpu.VMEM((2, 1, _D), jnp.float32),
            pltpu.VMEM((2, 1, _D), jnp.float32),
            pltpu.VMEM((2, 1, _D), jnp.float32),
            pltpu.SemaphoreType.DMA((6, 2)),
        ],
    )(x, gate_W1, gate_b1.reshape(1, -1), gate_W2, gate_b2.reshape(1, -1),
      We1, be1.reshape(_E, 1, _F), We2, be2.reshape(_E, 1, _D),
      ln_w.reshape(_E, 1, _D), ln_b.reshape(_E, 1, _D))
    return out.reshape(_D), probs.reshape(_E)
